# baseline (speedup 1.0000x reference)
import jax
import jax.numpy as jnp
from jax.experimental import pallas as pl


_H = 256
_W = 256
_K = 19


def _fused_kernel(hm_ref, paf_ref, out_ref):
    j = pl.program_id(1)

    @pl.when(j == 0)
    def _peaks():
        t = hm_ref[0].reshape(_K, _H, _W)
        ninfr = jnp.full((_K, 1, _W), -jnp.inf, dtype=t.dtype)
        ninfc = jnp.full((_K, _H, 1), -jnp.inf, dtype=t.dtype)
        nxt_col = jnp.concatenate([t[:, :, 1:], ninfc], axis=2)
        prv_col = jnp.concatenate([ninfc, t[:, :, :-1]], axis=2)
        nxt_row = jnp.concatenate([t[:, 1:, :], ninfr], axis=1)
        prv_row = jnp.concatenate([ninfr, t[:, :-1, :]], axis=1)

        nmax = jnp.maximum(jnp.maximum(nxt_col, prv_col),
                           jnp.maximum(nxt_row, prv_row))
        peak = (t > nmax) & (t >= 0.1)
        out_ref[0, 0] = jnp.where(peak, t, 0.0).reshape(_K * _H, _W)

    @pl.when(j == 1)
    def _mags():
        p = paf_ref[0].reshape(_K, 2, _H, _W)
        px = p[:, 0]
        py = p[:, 1]
        mag = jnp.sqrt(px * px + py * py + 1e-12)
        out_ref[0, 0] = mag.reshape(_K * _H, _W)


def kernel(heatmap2d, paf2d):
    B, K, H, W = heatmap2d.shape
    hm3 = heatmap2d.reshape(B, K * H, W)
    paf3 = paf2d.reshape(B, 2 * K * H, W)

    out = pl.pallas_call(
        _fused_kernel,
        grid=(B, 2),
        in_specs=[
            pl.BlockSpec((1, K * H, W), lambda b, j: (b, 0, 0)),
            pl.BlockSpec((1, 2 * K * H, W), lambda b, j: (b, 0, 0)),
        ],
        out_specs=pl.BlockSpec((1, 1, K * H, W), lambda b, j: (b, j, 0, 0)),
        out_shape=jax.ShapeDtypeStruct((B, 2, K * H, W), heatmap2d.dtype),
    )(hm3, paf3)

    return out.reshape(B, 2 * K, H, W)


# speedup vs baseline: 1.5540x; 1.5540x over previous
import jax
import jax.numpy as jnp
from jax.experimental import pallas as pl
from jax.experimental.pallas import tpu as pltpu


_H = 256
_W = 256
_K = 19


def _fused_kernel(hm_hbm, paf_hbm, out_ref, hm_buf, paf_buf, hm_sem, paf_sem):
    b = pl.program_id(0)
    j = pl.program_id(1)
    nb = pl.num_programs(0)
    cur = b % 2
    nxt = (b + 1) % 2

    @pl.when(j == 0)
    def _peaks():
        @pl.when(b == 0)
        def _prologue():
            pltpu.make_async_copy(hm_hbm.at[0], hm_buf.at[0], hm_sem.at[0]).start()
            pltpu.make_async_copy(paf_hbm.at[0], paf_buf.at[0], paf_sem.at[0]).start()

        @pl.when(b + 1 < nb)
        def _prefetch_hm():
            pltpu.make_async_copy(
                hm_hbm.at[b + 1], hm_buf.at[nxt], hm_sem.at[nxt]).start()

        pltpu.make_async_copy(hm_hbm.at[b], hm_buf.at[cur], hm_sem.at[cur]).wait()
        t = hm_buf[pl.ds(cur, 1)][0].reshape(_K, _H, _W)

        ninfr = jnp.full((_K, 1, _W), -jnp.inf, dtype=t.dtype)
        ninfc = jnp.full((_K, _H, 1), -jnp.inf, dtype=t.dtype)
        nxt_col = jnp.concatenate([t[:, :, 1:], ninfc], axis=2)
        prv_col = jnp.concatenate([ninfc, t[:, :, :-1]], axis=2)
        nxt_row = jnp.concatenate([t[:, 1:, :], ninfr], axis=1)
        prv_row = jnp.concatenate([ninfr, t[:, :-1, :]], axis=1)

        nmax = jnp.maximum(jnp.maximum(nxt_col, prv_col),
                           jnp.maximum(nxt_row, prv_row))
        peak = (t > nmax) & (t >= 0.1)
        out_ref[0, 0] = jnp.where(peak, t, 0.0).reshape(_K * _H, _W)

    @pl.when(j == 1)
    def _mags():
        @pl.when(b + 1 < nb)
        def _prefetch_paf():
            pltpu.make_async_copy(
                paf_hbm.at[b + 1], paf_buf.at[nxt], paf_sem.at[nxt]).start()

        pltpu.make_async_copy(paf_hbm.at[b], paf_buf.at[cur], paf_sem.at[cur]).wait()
        p = paf_buf[pl.ds(cur, 1)][0].reshape(_K, 2, _H, _W)
        px = p[:, 0]
        py = p[:, 1]
        mag = jnp.sqrt(px * px + py * py + 1e-12)
        out_ref[0, 0] = mag.reshape(_K * _H, _W)


def kernel(heatmap2d, paf2d):
    B, K, H, W = heatmap2d.shape
    hm3 = heatmap2d.reshape(B, K * H, W)
    paf3 = paf2d.reshape(B, 2 * K * H, W)

    out = pl.pallas_call(
        _fused_kernel,
        grid=(B, 2),
        in_specs=[
            pl.BlockSpec(memory_space=pl.ANY),
            pl.BlockSpec(memory_space=pl.ANY),
        ],
        out_specs=pl.BlockSpec((1, 1, K * H, W), lambda b, j: (b, j, 0, 0)),
        out_shape=jax.ShapeDtypeStruct((B, 2, K * H, W), heatmap2d.dtype),
        scratch_shapes=[
            pltpu.VMEM((2, K * H, W), jnp.float32),
            pltpu.VMEM((2, 2 * K * H, W), jnp.float32),
            pltpu.SemaphoreType.DMA((2,)),
            pltpu.SemaphoreType.DMA((2,)),
        ],
    )(hm3, paf3)

    return out.reshape(B, 2 * K, H, W)
